# Initial kernel scaffold; baseline (speedup 1.0000x reference)
import jax
import jax.numpy as jnp
from jax.experimental import pallas as pl


def kernel(graph, x, shuffled_index, W1, b1, W2, b2):
    raise NotImplementedError("write your pallas kernel here")


# speedup vs baseline: 6.9992x; 6.9992x over previous
import functools

import jax
import jax.numpy as jnp
from jax import lax
from jax.experimental import pallas as pl
from jax.experimental.pallas import tpu as pltpu
from jax.experimental.pallas import tpu_sc as plsc

N = 10000
D = 128
E = 320000

NC = 2
NS = 16
NW = NC * NS
EPW = E // NW
C = 80
NCHUNK = EPW // C
GC = N // C
KMAX = (GC + NS - 1) // NS

_mesh = plsc.VectorSubcoreMesh(core_axis_name="c", subcore_axis_name="s")


def _zero_rows(ref, nrows, ncols):
    def body(i, _):
        for l in range(ncols // 16):
            ref[i, pl.ds(l * 16, 16)] = jnp.zeros((16,), jnp.float32)
        return 0
    lax.fori_loop(0, nrows, body, 0, unroll=False)


def _segsum_body(with_deg, *refs):
    if with_deg:
        (src_hbm, dst_hbm, x_hbm, out_hbm, deg0_hbm, deg1_hbm,
         src_all, dst_all, dst_chunk, rows, ones1d, zbuf, zbufd, acc, dacc) = refs
    else:
        (src_hbm, dst_hbm, x_hbm, out_hbm,
         src_all, dst_all, dst_chunk, rows, zbuf, acc) = refs
    c = lax.axis_index("c")
    s = lax.axis_index("s")
    wid = c * NS + s

    _zero_rows(zbuf, C, D)
    if with_deg:
        def zb(i, _):
            zbufd[pl.ds(i * 16, 16)] = jnp.zeros((16,), jnp.float32)
            return 0
        lax.fori_loop(0, C // 16, zb, 0, unroll=False)
        def ob(i, _):
            ones1d[pl.ds(i * 16, 16)] = jnp.ones((16,), jnp.float32)
            return 0
        lax.fori_loop(0, C // 16, ob, 0, unroll=False)
    for k in range(KMAX):
        j = s + k * NS
        @pl.when(j < GC)
        def _():
            pltpu.sync_copy(zbuf, acc.at[pl.ds(j * C, C)])
            if with_deg:
                pltpu.sync_copy(zbufd, dacc.at[pl.ds(j * C, C)])
    plsc.subcore_barrier()

    pltpu.sync_copy(src_hbm.at[pl.ds(wid * EPW, EPW)], src_all)
    pltpu.sync_copy(dst_hbm.at[pl.ds(wid * EPW, EPW)], dst_all)

    def chunk(j, _):
        base = j * C
        for l in range(C // 16):
            dst_chunk[pl.ds(l * 16, 16)] = dst_all[pl.ds(base + l * 16, 16)]
        pltpu.sync_copy(x_hbm.at[src_all.at[pl.ds(base, C)]], rows)
        pltpu.sync_copy(rows, acc.at[dst_chunk], add=True)
        if with_deg:
            pltpu.sync_copy(ones1d, dacc.at[dst_chunk], add=True)
        return 0
    lax.fori_loop(0, NCHUNK, chunk, 0, unroll=False)
    plsc.subcore_barrier()

    for k in range(KMAX):
        j = s + k * NS
        @pl.when(j < GC)
        def _():
            pltpu.sync_copy(acc.at[pl.ds(j * C, C)],
                            out_hbm.at[c, pl.ds(j * C, C)])
        if with_deg:
            @pl.when(jnp.logical_and(j < GC, c == 0))
            def _():
                pltpu.sync_copy(dacc.at[pl.ds(j * C, C)], zbufd)
                pltpu.sync_copy(zbufd, deg0_hbm.at[pl.ds(j * C, C)])
            @pl.when(jnp.logical_and(j < GC, c == 1))
            def _():
                pltpu.sync_copy(dacc.at[pl.ds(j * C, C)], zbufd)
                pltpu.sync_copy(zbufd, deg1_hbm.at[pl.ds(j * C, C)])


_segsum_deg = pl.kernel(
    functools.partial(_segsum_body, True),
    out_type=(jax.ShapeDtypeStruct((NC, N, D), jnp.float32),
              jax.ShapeDtypeStruct((N,), jnp.float32),
              jax.ShapeDtypeStruct((N,), jnp.float32)),
    mesh=_mesh,
    scratch_types=[
        pltpu.VMEM((EPW,), jnp.int32),
        pltpu.VMEM((EPW,), jnp.int32),
        pltpu.VMEM((C,), jnp.int32),
        pltpu.VMEM((C, D), jnp.float32),
        pltpu.VMEM((C,), jnp.float32),
        pltpu.VMEM((C, D), jnp.float32),
        pltpu.VMEM((C,), jnp.float32),
        pltpu.VMEM_SHARED((N, D), jnp.float32),
        pltpu.VMEM_SHARED((N,), jnp.float32),
    ],
)

_segsum = pl.kernel(
    functools.partial(_segsum_body, False),
    out_type=jax.ShapeDtypeStruct((NC, N, D), jnp.float32),
    mesh=_mesh,
    scratch_types=[
        pltpu.VMEM((EPW,), jnp.int32),
        pltpu.VMEM((EPW,), jnp.int32),
        pltpu.VMEM((C,), jnp.int32),
        pltpu.VMEM((C, D), jnp.float32),
        pltpu.VMEM((C, D), jnp.float32),
        pltpu.VMEM_SHARED((N, D), jnp.float32),
    ],
)


def _gather_body(h_hbm, shuf_hbm, out_hbm, idx_v, rows):
    c = lax.axis_index("c")
    s = lax.axis_index("s")
    wid = c * NS + s
    for k in range((GC + NW - 1) // NW):
        j = wid + k * NW
        @pl.when(j < GC)
        def _():
            pltpu.sync_copy(shuf_hbm.at[pl.ds(j * C, C)], idx_v)
            pltpu.sync_copy(h_hbm.at[idx_v], rows)
            pltpu.sync_copy(rows, out_hbm.at[pl.ds(j * C, C)])


_gather = pl.kernel(
    _gather_body,
    out_type=jax.ShapeDtypeStruct((N, D), jnp.float32),
    mesh=_mesh,
    scratch_types=[
        pltpu.VMEM((C,), jnp.int32),
        pltpu.VMEM((C, D), jnp.float32),
    ],
)


def _layer_tc(relu, hp_ref, d0_ref, d1_ref, h_ref, w_ref, b_ref, out_ref):
    hs = hp_ref[0] + hp_ref[1] + h_ref[...]
    deg = (d0_ref[...] + d1_ref[...] + 1.0).reshape(N, 1)
    hn = hs / deg
    y = jnp.dot(hn, w_ref[...], preferred_element_type=jnp.float32,
                precision=lax.Precision.HIGHEST) + b_ref[...].reshape(1, D)
    out_ref[...] = jnp.maximum(y, 0.0) if relu else y


def _decoder_tc(h_ref, g_ref, out_ref):
    a = h_ref[...]
    b = g_ref[...]
    num = jnp.sum(a * b, axis=1)
    na = jnp.sum(a * a, axis=1)
    nb = jnp.sum(b * b, axis=1)
    denom = jnp.maximum(jnp.sqrt(na) * jnp.sqrt(nb), 1e-8)
    out_ref[...] = num / denom


def _layer(relu, hsum_p, deg0, deg1, h, W, b):
    return pl.pallas_call(
        functools.partial(_layer_tc, relu),
        out_shape=jax.ShapeDtypeStruct((N, D), jnp.float32),
    )(hsum_p, deg0, deg1, h, W, b)


def _decoder(h2, g):
    return pl.pallas_call(
        _decoder_tc,
        out_shape=jax.ShapeDtypeStruct((N,), jnp.float32),
    )(h2, g)


def kernel(graph, x, shuffled_index, W1, b1, W2, b2):
    src = graph[0]
    dst = graph[1]
    hs1, dg0, dg1 = _segsum_deg(src, dst, x)
    h1 = _layer(True, hs1, dg0, dg1, x, W1, b1)
    hs2 = _segsum(src, dst, h1)
    h2 = _layer(False, hs2, dg0, dg1, h1, W2, b2)
    g = _gather(h2, shuffled_index)
    dec = _decoder(h2, g)
    return (h2, dec)


# speedup vs baseline: 9.0064x; 1.2868x over previous
import functools

import jax
import jax.numpy as jnp
from jax import lax
from jax.experimental import pallas as pl
from jax.experimental.pallas import tpu as pltpu
from jax.experimental.pallas import tpu_sc as plsc

N = 10000
D = 128
E = 320000

NC = 2
NS = 16
NW = NC * NS
EPW = E // NW
C = 80
NCHUNK = EPW // C
GC = N // C
KMAX = (GC + NS - 1) // NS

_mesh = plsc.VectorSubcoreMesh(core_axis_name="c", subcore_axis_name="s")


def _segsum_body(with_deg, *refs):
    if with_deg:
        (src_hbm, dst_hbm, x_hbm, out_hbm, deg0_hbm, deg1_hbm,
         src_all, dst_all, dst2, rows2, gsem, ones1d, zbufd, acc, dacc) = refs
    else:
        (src_hbm, dst_hbm, x_hbm, out_hbm,
         src_all, dst_all, dst2, rows2, gsem, acc) = refs
    c = lax.axis_index("c")
    s = lax.axis_index("s")
    wid = c * NS + s

    def zrow(i, _):
        for l in range(D // 16):
            rows2[0, i, pl.ds(l * 16, 16)] = jnp.zeros((16,), jnp.float32)
        return 0
    lax.fori_loop(0, C, zrow, 0, unroll=False)
    if with_deg:
        def zb(i, _):
            zbufd[pl.ds(i * 16, 16)] = jnp.zeros((16,), jnp.float32)
            return 0
        lax.fori_loop(0, C // 16, zb, 0, unroll=False)
        def ob(i, _):
            ones1d[pl.ds(i * 16, 16)] = jnp.ones((16,), jnp.float32)
            return 0
        lax.fori_loop(0, C // 16, ob, 0, unroll=False)
    for k in range(KMAX):
        j = s + k * NS
        @pl.when(j < GC)
        def _():
            pltpu.sync_copy(rows2.at[0], acc.at[pl.ds(j * C, C)])
            if with_deg:
                pltpu.sync_copy(zbufd, dacc.at[pl.ds(j * C, C)])
    plsc.subcore_barrier()

    pltpu.sync_copy(src_hbm.at[pl.ds(wid * EPW, EPW)], src_all)
    pltpu.sync_copy(dst_hbm.at[pl.ds(wid * EPW, EPW)], dst_all)

    def stage(j, b):
        for l in range(C // 16):
            dst2[b, pl.ds(l * 16, 16)] = dst_all[pl.ds(j * C + l * 16, 16)]
        pltpu.async_copy(x_hbm.at[src_all.at[pl.ds(j * C, C)]],
                         rows2.at[b], gsem)

    stage(0, 0)

    def chunk(j, _):
        b = lax.rem(j, 2)
        pltpu.make_async_copy(x_hbm.at[src_all.at[pl.ds(j * C, C)]],
                              rows2.at[b], gsem).wait()
        @pl.when(j + 1 < NCHUNK)
        def _():
            stage(j + 1, 1 - b)
        pltpu.sync_copy(rows2.at[b], acc.at[dst2.at[b]], add=True)
        if with_deg:
            pltpu.sync_copy(ones1d, dacc.at[dst2.at[b]], add=True)
        return 0
    lax.fori_loop(0, NCHUNK, chunk, 0, unroll=False)
    plsc.subcore_barrier()

    for k in range(KMAX):
        j = s + k * NS
        @pl.when(j < GC)
        def _():
            pltpu.sync_copy(acc.at[pl.ds(j * C, C)],
                            out_hbm.at[c, pl.ds(j * C, C)])
        if with_deg:
            @pl.when(jnp.logical_and(j < GC, c == 0))
            def _():
                pltpu.sync_copy(dacc.at[pl.ds(j * C, C)], zbufd)
                pltpu.sync_copy(zbufd, deg0_hbm.at[pl.ds(j * C, C)])
            @pl.when(jnp.logical_and(j < GC, c == 1))
            def _():
                pltpu.sync_copy(dacc.at[pl.ds(j * C, C)], zbufd)
                pltpu.sync_copy(zbufd, deg1_hbm.at[pl.ds(j * C, C)])


_segsum_deg = pl.kernel(
    functools.partial(_segsum_body, True),
    out_type=(jax.ShapeDtypeStruct((NC, N, D), jnp.float32),
              jax.ShapeDtypeStruct((N,), jnp.float32),
              jax.ShapeDtypeStruct((N,), jnp.float32)),
    mesh=_mesh,
    scratch_types=[
        pltpu.VMEM((EPW,), jnp.int32),
        pltpu.VMEM((EPW,), jnp.int32),
        pltpu.VMEM((2, C), jnp.int32),
        pltpu.VMEM((2, C, D), jnp.float32),
        pltpu.SemaphoreType.DMA,
        pltpu.VMEM((C,), jnp.float32),
        pltpu.VMEM((C,), jnp.float32),
        pltpu.VMEM_SHARED((N, D), jnp.float32),
        pltpu.VMEM_SHARED((N,), jnp.float32),
    ],
)

_segsum = pl.kernel(
    functools.partial(_segsum_body, False),
    out_type=jax.ShapeDtypeStruct((NC, N, D), jnp.float32),
    mesh=_mesh,
    scratch_types=[
        pltpu.VMEM((EPW,), jnp.int32),
        pltpu.VMEM((EPW,), jnp.int32),
        pltpu.VMEM((2, C), jnp.int32),
        pltpu.VMEM((2, C, D), jnp.float32),
        pltpu.SemaphoreType.DMA,
        pltpu.VMEM_SHARED((N, D), jnp.float32),
    ],
)


def _gather_body(h_hbm, shuf_hbm, out_hbm, idx_v, rows):
    c = lax.axis_index("c")
    s = lax.axis_index("s")
    wid = c * NS + s
    for k in range((GC + NW - 1) // NW):
        j = wid + k * NW
        @pl.when(j < GC)
        def _():
            pltpu.sync_copy(shuf_hbm.at[pl.ds(j * C, C)], idx_v)
            pltpu.sync_copy(h_hbm.at[idx_v], rows)
            pltpu.sync_copy(rows, out_hbm.at[pl.ds(j * C, C)])


_gather = pl.kernel(
    _gather_body,
    out_type=jax.ShapeDtypeStruct((N, D), jnp.float32),
    mesh=_mesh,
    scratch_types=[
        pltpu.VMEM((C,), jnp.int32),
        pltpu.VMEM((C, D), jnp.float32),
    ],
)


def _layer_tc(relu, hp_ref, d0_ref, d1_ref, h_ref, w_ref, b_ref, out_ref):
    hs = hp_ref[0] + hp_ref[1] + h_ref[...]
    deg = (d0_ref[...] + d1_ref[...] + 1.0).reshape(N, 1)
    hn = hs / deg
    y = jnp.dot(hn, w_ref[...], preferred_element_type=jnp.float32,
                precision=lax.Precision.HIGHEST) + b_ref[...].reshape(1, D)
    out_ref[...] = jnp.maximum(y, 0.0) if relu else y


def _decoder_tc(h_ref, g_ref, out_ref):
    a = h_ref[...]
    b = g_ref[...]
    num = jnp.sum(a * b, axis=1)
    na = jnp.sum(a * a, axis=1)
    nb = jnp.sum(b * b, axis=1)
    denom = jnp.maximum(jnp.sqrt(na) * jnp.sqrt(nb), 1e-8)
    out_ref[...] = num / denom


def _layer(relu, hsum_p, deg0, deg1, h, W, b):
    return pl.pallas_call(
        functools.partial(_layer_tc, relu),
        out_shape=jax.ShapeDtypeStruct((N, D), jnp.float32),
    )(hsum_p, deg0, deg1, h, W, b)


def _decoder(h2, g):
    return pl.pallas_call(
        _decoder_tc,
        out_shape=jax.ShapeDtypeStruct((N,), jnp.float32),
    )(h2, g)


def kernel(graph, x, shuffled_index, W1, b1, W2, b2):
    src = graph[0]
    dst = graph[1]
    hs1, dg0, dg1 = _segsum_deg(src, dst, x)
    h1 = _layer(True, hs1, dg0, dg1, x, W1, b1)
    hs2 = _segsum(src, dst, h1)
    h2 = _layer(False, hs2, dg0, dg1, h1, W2, b2)
    g = _gather(h2, shuffled_index)
    dec = _decoder(h2, g)
    return (h2, dec)


# speedup vs baseline: 10.5880x; 1.1756x over previous
import functools

import jax
import jax.numpy as jnp
from jax import lax
from jax.experimental import pallas as pl
from jax.experimental.pallas import tpu as pltpu
from jax.experimental.pallas import tpu_sc as plsc

N = 10000
D = 128
E = 320000

NC = 2
NS = 16
NW = NC * NS
C = 80
GC = N // C
KMAX = (GC + NS - 1) // NS
CE = 128
TOT = E // CE
KE = (TOT + NW - 1) // NW

_mesh = plsc.VectorSubcoreMesh(core_axis_name="c", subcore_axis_name="s")


def _segsum_body(with_deg, *refs):
    if with_deg:
        (src_hbm, dst_hbm, x_hbm, out_hbm, deg0_hbm, deg1_hbm,
         sidx, didx, rows2, isem, gsem, ssem, ones1d, zbufd, acc, dacc) = refs
    else:
        (src_hbm, dst_hbm, x_hbm, out_hbm,
         sidx, didx, rows2, isem, gsem, ssem, acc) = refs
    c = lax.axis_index("c")
    s = lax.axis_index("s")
    wid = c * NS + s

    def zrow(i, _):
        for l in range(D // 16):
            rows2[0, i, pl.ds(l * 16, 16)] = jnp.zeros((16,), jnp.float32)
        return 0
    lax.fori_loop(0, C, zrow, 0, unroll=False)
    if with_deg:
        def zb(i, _):
            zbufd[pl.ds(i * 16, 16)] = jnp.zeros((16,), jnp.float32)
            return 0
        lax.fori_loop(0, C // 16, zb, 0, unroll=False)
        def ob(i, _):
            ones1d[pl.ds(i * 16, 16)] = jnp.ones((16,), jnp.float32)
            return 0
        lax.fori_loop(0, CE // 16, ob, 0, unroll=False)
    for k in range(KMAX):
        j = s + k * NS
        @pl.when(j < GC)
        def _():
            pltpu.sync_copy(rows2.at[0, pl.ds(0, C)], acc.at[pl.ds(j * C, C)])
            if with_deg:
                pltpu.sync_copy(zbufd, dacc.at[pl.ds(j * C, C)])
    plsc.subcore_barrier()

    pltpu.sync_copy(src_hbm.at[pl.ds(wid * CE, CE)], sidx.at[0])
    pltpu.sync_copy(dst_hbm.at[pl.ds(wid * CE, CE)], didx.at[0])
    pltpu.async_copy(src_hbm.at[pl.ds((wid + NW) * CE, CE)], sidx.at[1], isem)
    pltpu.async_copy(dst_hbm.at[pl.ds((wid + NW) * CE, CE)], didx.at[1], isem)
    pltpu.async_copy(x_hbm.at[sidx.at[0]], rows2.at[0], gsem)

    def chunk(k, _):
        j = wid + k * NW
        b = lax.rem(k, 2)
        b1 = lax.rem(k + 1, 2)
        sl = lax.rem(k, 3)
        sl1 = lax.rem(k + 1, 3)
        sl2 = lax.rem(k + 2, 3)
        @pl.when(j < TOT)
        def _():
            @pl.when(k >= 1)
            def _():
                pltpu.make_async_copy(
                    rows2.at[b1], acc.at[didx.at[lax.rem(k + 2, 3)]],
                    ssem).wait()
            @pl.when(j + 2 * NW < TOT)
            def _():
                pltpu.async_copy(src_hbm.at[pl.ds((j + 2 * NW) * CE, CE)],
                                 sidx.at[sl2], isem)
                pltpu.async_copy(dst_hbm.at[pl.ds((j + 2 * NW) * CE, CE)],
                                 didx.at[sl2], isem)
            pltpu.make_async_copy(x_hbm.at[sidx.at[sl]], rows2.at[b],
                                  gsem).wait()
            @pl.when(j + NW < TOT)
            def _():
                pltpu.make_async_copy(src_hbm.at[pl.ds((j + NW) * CE, CE)],
                                      sidx.at[sl1], isem).wait()
                pltpu.make_async_copy(dst_hbm.at[pl.ds((j + NW) * CE, CE)],
                                      didx.at[sl1], isem).wait()
                pltpu.async_copy(x_hbm.at[sidx.at[sl1]], rows2.at[b1], gsem)
            pltpu.async_copy(rows2.at[b], acc.at[didx.at[sl]], ssem, add=True)
            if with_deg:
                pltpu.sync_copy(ones1d, dacc.at[didx.at[sl]], add=True)
        return 0
    lax.fori_loop(0, KE, chunk, 0, unroll=False)
    pltpu.make_async_copy(rows2.at[0], acc.at[didx.at[0]], ssem).wait()
    plsc.subcore_barrier()

    for k in range(KMAX):
        j = s + k * NS
        @pl.when(j < GC)
        def _():
            pltpu.sync_copy(acc.at[pl.ds(j * C, C)],
                            out_hbm.at[c, pl.ds(j * C, C)])
        if with_deg:
            @pl.when(jnp.logical_and(j < GC, c == 0))
            def _():
                pltpu.sync_copy(dacc.at[pl.ds(j * C, C)], zbufd)
                pltpu.sync_copy(zbufd, deg0_hbm.at[pl.ds(j * C, C)])
            @pl.when(jnp.logical_and(j < GC, c == 1))
            def _():
                pltpu.sync_copy(dacc.at[pl.ds(j * C, C)], zbufd)
                pltpu.sync_copy(zbufd, deg1_hbm.at[pl.ds(j * C, C)])


_segsum_deg = pl.kernel(
    functools.partial(_segsum_body, True),
    out_type=(jax.ShapeDtypeStruct((NC, N, D), jnp.float32),
              jax.ShapeDtypeStruct((N,), jnp.float32),
              jax.ShapeDtypeStruct((N,), jnp.float32)),
    mesh=_mesh,
    scratch_types=[
        pltpu.VMEM((3, CE), jnp.int32),
        pltpu.VMEM((3, CE), jnp.int32),
        pltpu.VMEM((2, CE, D), jnp.float32),
        pltpu.SemaphoreType.DMA,
        pltpu.SemaphoreType.DMA,
        pltpu.SemaphoreType.DMA,
        pltpu.VMEM((CE,), jnp.float32),
        pltpu.VMEM((C,), jnp.float32),
        pltpu.VMEM_SHARED((N, D), jnp.float32),
        pltpu.VMEM_SHARED((N,), jnp.float32),
    ],
)

_segsum = pl.kernel(
    functools.partial(_segsum_body, False),
    out_type=jax.ShapeDtypeStruct((NC, N, D), jnp.float32),
    mesh=_mesh,
    scratch_types=[
        pltpu.VMEM((3, CE), jnp.int32),
        pltpu.VMEM((3, CE), jnp.int32),
        pltpu.VMEM((2, CE, D), jnp.float32),
        pltpu.SemaphoreType.DMA,
        pltpu.SemaphoreType.DMA,
        pltpu.SemaphoreType.DMA,
        pltpu.VMEM_SHARED((N, D), jnp.float32),
    ],
)


def _gather_body(h_hbm, shuf_hbm, out_hbm, idx_v, rows):
    c = lax.axis_index("c")
    s = lax.axis_index("s")
    wid = c * NS + s
    for k in range((GC + NW - 1) // NW):
        j = wid + k * NW
        @pl.when(j < GC)
        def _():
            pltpu.sync_copy(shuf_hbm.at[pl.ds(j * C, C)], idx_v)
            pltpu.sync_copy(h_hbm.at[idx_v], rows)
            pltpu.sync_copy(rows, out_hbm.at[pl.ds(j * C, C)])


_gather = pl.kernel(
    _gather_body,
    out_type=jax.ShapeDtypeStruct((N, D), jnp.float32),
    mesh=_mesh,
    scratch_types=[
        pltpu.VMEM((C,), jnp.int32),
        pltpu.VMEM((C, D), jnp.float32),
    ],
)


def _layer_tc(relu, hp_ref, d0_ref, d1_ref, h_ref, w_ref, b_ref, out_ref):
    hs = hp_ref[0] + hp_ref[1] + h_ref[...]
    deg = (d0_ref[...] + d1_ref[...] + 1.0).reshape(N, 1)
    hn = hs / deg
    y = jnp.dot(hn, w_ref[...], preferred_element_type=jnp.float32,
                precision=lax.Precision.HIGHEST) + b_ref[...].reshape(1, D)
    out_ref[...] = jnp.maximum(y, 0.0) if relu else y


def _decoder_tc(h_ref, g_ref, out_ref):
    a = h_ref[...]
    b = g_ref[...]
    num = jnp.sum(a * b, axis=1)
    na = jnp.sum(a * a, axis=1)
    nb = jnp.sum(b * b, axis=1)
    denom = jnp.maximum(jnp.sqrt(na) * jnp.sqrt(nb), 1e-8)
    out_ref[...] = num / denom


def _layer(relu, hsum_p, deg0, deg1, h, W, b):
    return pl.pallas_call(
        functools.partial(_layer_tc, relu),
        out_shape=jax.ShapeDtypeStruct((N, D), jnp.float32),
    )(hsum_p, deg0, deg1, h, W, b)


def _decoder(h2, g):
    return pl.pallas_call(
        _decoder_tc,
        out_shape=jax.ShapeDtypeStruct((N,), jnp.float32),
    )(h2, g)


def kernel(graph, x, shuffled_index, W1, b1, W2, b2):
    src = graph[0]
    dst = graph[1]
    hs1, dg0, dg1 = _segsum_deg(src, dst, x)
    h1 = _layer(True, hs1, dg0, dg1, x, W1, b1)
    hs2 = _segsum(src, dst, h1)
    h2 = _layer(False, hs2, dg0, dg1, h1, W2, b2)
    g = _gather(h2, shuffled_index)
    dec = _decoder(h2, g)
    return (h2, dec)
